# baseline (speedup 1.0000x reference)
import functools

import jax
import jax.numpy as jnp
from jax.experimental import pallas as pl
from jax.experimental.pallas import tpu as pltpu

_SCALE = 1.0


def _fiota(shape, dim):
    return jax.lax.broadcasted_iota(jnp.int32, shape, dim).astype(jnp.float32)


def _route_body(x_ref, gw_ref, w_ref, e_ref):
    x = x_ref[...]
    logits = jax.lax.dot_general(x, gw_ref[...], (((1,), (1,)), ((), ())),
                                 preferred_element_type=jnp.float32)
    m = jnp.max(logits, axis=1, keepdims=True)
    p = jnp.exp(logits - m)
    sc = p / jnp.sum(p, axis=1, keepdims=True)
    bt, ne = sc.shape
    ie = _fiota((bt, ne), 1)
    m1 = jnp.max(sc, axis=1, keepdims=True)
    i1 = jnp.min(jnp.where(sc == m1, ie, float(ne)), axis=1, keepdims=True)
    sc2 = jnp.where(ie == i1, -1.0, sc)
    m2 = jnp.max(sc2, axis=1, keepdims=True)
    i2 = jnp.min(jnp.where(sc2 == m2, ie, float(ne)), axis=1, keepdims=True)
    w_ref[...] = jnp.where(ie == 0.0, m1 * _SCALE,
                           jnp.where(ie == 1.0, m2 * _SCALE, 0.0))
    e_ref[...] = jnp.where(ie == 0.0, i1, jnp.where(ie == 1.0, i2, 0.0))


def _count_body(e_ref, rank_ref, meta_ref, c_ref, *, nt, bt_tile, ntiles):
    b = pl.program_id(0)
    bt, ne = e_ref.shape

    @pl.when(b == 0)
    def _():
        c_ref[...] = jnp.zeros_like(c_ref)

    ie = _fiota((bt, ne), 1)
    o1 = (ie == e_ref[:, 0:1]).astype(jnp.float32)
    o2 = (ie == e_ref[:, 1:2]).astype(jnp.float32)
    s = o1 + o2
    r = _fiota((bt, bt), 0)
    cc = _fiota((bt, bt), 1)
    tri = (r > cc).astype(jnp.float32)
    cum = jnp.dot(tri, s, preferred_element_type=jnp.float32)
    crow = c_ref[0:1, :ne]
    rank1 = jnp.sum(o1 * (cum + crow), axis=1, keepdims=True)
    rank2 = jnp.sum(o2 * (cum + o1 + crow), axis=1, keepdims=True)
    rank_ref[...] = jnp.where(ie == 0.0, rank1,
                              jnp.where(ie == 1.0, rank2, 0.0))
    cnew = crow + jnp.sum(s, axis=0, keepdims=True)
    c_ref[0:1, :ne] = cnew

    @pl.when(b == nt - 1)
    def _():
        pc = jnp.ceil(cnew / bt_tile) * bt_tile
        iu = _fiota((ne, ne), 0)
        ju = _fiota((ne, ne), 1)
        ut = (iu < ju).astype(jnp.float32)
        offs = jnp.dot(pc, ut, preferred_element_type=jnp.float32)
        ends = offs + pc
        rows, cols = meta_ref.shape
        icol = _fiota((1, cols), 1)
        te = jnp.zeros((1, cols), jnp.float32)
        for eidx in range(ne):
            te = te + (icol * bt_tile >= ends[0, eidx]).astype(jnp.float32)
        te = jnp.minimum(te, float(ne - 1))
        irow = _fiota((rows, cols), 0)
        meta_ref[...] = jnp.where(
            irow == 0.0, jnp.pad(cnew, ((0, 0), (0, cols - ne))),
            jnp.where(irow == 1.0, jnp.pad(offs, ((0, 0), (0, cols - ne))),
                      jnp.where(irow == 2.0, te, 0.0)))


def _pos_from(rank, eidx, meta, ne):
    offs = meta[1:2, :ne]
    ie = _fiota(eidx.shape, 1)
    o1 = (ie == eidx[:, 0:1]).astype(jnp.float32)
    o2 = (ie == eidx[:, 1:2]).astype(jnp.float32)
    pos1 = rank[:, 0] + jnp.sum(o1 * offs, axis=1)
    pos2 = rank[:, 1] + jnp.sum(o2 * offs, axis=1)
    return pos1, pos2


def _gather_body(rank_ref, e_ref, meta_ref, x_ref, xs_ref, acc_ref,
                 *, bt_tile, ne, nt):
    i = pl.program_id(0)
    j = pl.program_id(1)
    pos1, pos2 = _pos_from(rank_ref[...], e_ref[...], meta_ref[...], ne)
    bt = rank_ref.shape[0]
    pj = (i * bt_tile
          + _fiota((bt_tile, bt), 0))
    m = ((pos1[None, :] == pj).astype(jnp.float32)
         + (pos2[None, :] == pj).astype(jnp.float32))
    m = jnp.minimum(m, 1.0)
    y = jnp.dot(m, x_ref[...], preferred_element_type=jnp.float32)

    @pl.when(j == 0)
    def _():
        acc_ref[...] = jnp.zeros_like(acc_ref)

    acc_ref[...] += y

    @pl.when(j == nt - 1)
    def _():
        xs_ref[...] = acc_ref[...]


def _gmm_body(te_ref, xs_ref, wg_ref, wu_ref, wd_ref, ys_ref, acc_ref, *, nf):
    f = pl.program_id(1)
    xs = xs_ref[...]
    g = jax.lax.dot_general(xs, wg_ref[0], (((1,), (1,)), ((), ())),
                            preferred_element_type=jnp.float32)
    u = jax.lax.dot_general(xs, wu_ref[0], (((1,), (1,)), ((), ())),
                            preferred_element_type=jnp.float32)
    h = (g * jax.lax.logistic(g)) * u
    y = jax.lax.dot_general(h, wd_ref[0], (((1,), (1,)), ((), ())),
                            preferred_element_type=jnp.float32)

    @pl.when(f == 0)
    def _():
        acc_ref[...] = jnp.zeros_like(acc_ref)

    acc_ref[...] += y

    @pl.when(f == nf - 1)
    def _():
        ys_ref[...] = acc_ref[...]


def _comb_body(rank_ref, e_ref, w_ref, meta_ref, ys_ref, out_ref, acc_ref,
               *, np_chunks, bt_tile, ne):
    p = pl.program_id(1)
    pos1, pos2 = _pos_from(rank_ref[...], e_ref[...], meta_ref[...], ne)
    w = w_ref[...]
    bt = rank_ref.shape[0]
    pj = (p * bt_tile
          + _fiota((bt, bt_tile), 1))
    c = (w[:, 0:1] * (pos1[:, None] == pj).astype(jnp.float32)
         + w[:, 1:2] * (pos2[:, None] == pj).astype(jnp.float32))
    y = jnp.dot(c, ys_ref[...], preferred_element_type=jnp.float32)

    @pl.when(p == 0)
    def _():
        acc_ref[...] = jnp.zeros_like(acc_ref)

    acc_ref[...] += y

    @pl.when(p == np_chunks - 1)
    def _():
        out_ref[...] = acc_ref[...]


def _shared_body(x_ref, wg_ref, wu_ref, wd_ref, comb_ref, out_ref, acc_ref,
                 *, nfs):
    f = pl.program_id(1)
    x = x_ref[...]
    g = jax.lax.dot_general(x, wg_ref[...], (((1,), (1,)), ((), ())),
                            preferred_element_type=jnp.float32)
    u = jax.lax.dot_general(x, wu_ref[...], (((1,), (1,)), ((), ())),
                            preferred_element_type=jnp.float32)
    h = (g * jax.lax.logistic(g)) * u
    y = jax.lax.dot_general(h, wd_ref[...], (((1,), (1,)), ((), ())),
                            preferred_element_type=jnp.float32)

    @pl.when(f == 0)
    def _():
        acc_ref[...] = comb_ref[...]

    acc_ref[...] += y

    @pl.when(f == nfs - 1)
    def _():
        out_ref[...] = acc_ref[...]


def _impl(hidden_states, gate_w, Wg, Wu, Wd, Wg_s, Wu_s, Wd_s,
          bt=256, bt_tile=256, bf=128, bfs=256, interpret=False):
    b, s, h = hidden_states.shape
    t = b * s
    ne, f, _ = Wg.shape
    fs = Wg_s.shape[0]
    x = hidden_states.reshape(t, h).astype(jnp.float32)
    nt = t // bt
    ntiles = (2 * t) // bt_tile + ne - 1
    p_tot = ntiles * bt_tile
    nf = f // bf
    nfs = fs // bfs

    wts, eidx = pl.pallas_call(
        _route_body,
        grid=(nt,),
        in_specs=[pl.BlockSpec((bt, h), lambda i: (i, 0)),
                  pl.BlockSpec((ne, h), lambda i: (0, 0))],
        out_specs=[pl.BlockSpec((bt, ne), lambda i: (i, 0)),
                   pl.BlockSpec((bt, ne), lambda i: (i, 0))],
        out_shape=[jax.ShapeDtypeStruct((t, ne), jnp.float32),
                   jax.ShapeDtypeStruct((t, ne), jnp.float32)],
        interpret=interpret,
    )(x, gate_w)

    rank, meta = pl.pallas_call(
        functools.partial(_count_body, nt=nt, bt_tile=bt_tile, ntiles=ntiles),
        grid=(nt,),
        in_specs=[pl.BlockSpec((bt, ne), lambda i: (i, 0))],
        out_specs=[pl.BlockSpec((bt, ne), lambda i: (i, 0)),
                   pl.BlockSpec((8, 128), lambda i: (0, 0))],
        out_shape=[jax.ShapeDtypeStruct((t, ne), jnp.float32),
                   jax.ShapeDtypeStruct((8, 128), jnp.float32)],
        scratch_shapes=[pltpu.VMEM((8, 128), jnp.float32)],
        interpret=interpret,
    )(eidx)

    tile_e = jnp.round(meta[2, :ntiles]).astype(jnp.int32)

    xs = pl.pallas_call(
        functools.partial(_gather_body, bt_tile=bt_tile, ne=ne, nt=nt),
        grid=(ntiles, nt),
        in_specs=[pl.BlockSpec((bt, ne), lambda i, j: (j, 0)),
                  pl.BlockSpec((bt, ne), lambda i, j: (j, 0)),
                  pl.BlockSpec((8, 128), lambda i, j: (0, 0)),
                  pl.BlockSpec((bt, h), lambda i, j: (j, 0))],
        out_specs=pl.BlockSpec((bt_tile, h), lambda i, j: (i, 0)),
        out_shape=jax.ShapeDtypeStruct((p_tot, h), jnp.float32),
        scratch_shapes=[pltpu.VMEM((bt_tile, h), jnp.float32)],
        interpret=interpret,
    )(rank, eidx, meta, x)

    ys = pl.pallas_call(
        functools.partial(_gmm_body, nf=nf),
        grid_spec=pltpu.PrefetchScalarGridSpec(
            num_scalar_prefetch=1,
            grid=(ntiles, nf),
            in_specs=[
                pl.BlockSpec((bt_tile, h), lambda i, j, te: (i, 0)),
                pl.BlockSpec((1, bf, h), lambda i, j, te: (te[i], j, 0)),
                pl.BlockSpec((1, bf, h), lambda i, j, te: (te[i], j, 0)),
                pl.BlockSpec((1, h, bf), lambda i, j, te: (te[i], 0, j)),
            ],
            out_specs=pl.BlockSpec((bt_tile, h), lambda i, j, te: (i, 0)),
            scratch_shapes=[pltpu.VMEM((bt_tile, h), jnp.float32)],
        ),
        out_shape=jax.ShapeDtypeStruct((p_tot, h), jnp.float32),
        interpret=interpret,
    )(tile_e, xs, Wg, Wu, Wd)

    comb = pl.pallas_call(
        functools.partial(_comb_body, np_chunks=ntiles, bt_tile=bt_tile,
                          ne=ne),
        grid=(nt, ntiles),
        in_specs=[pl.BlockSpec((bt, ne), lambda i, j: (i, 0)),
                  pl.BlockSpec((bt, ne), lambda i, j: (i, 0)),
                  pl.BlockSpec((bt, ne), lambda i, j: (i, 0)),
                  pl.BlockSpec((8, 128), lambda i, j: (0, 0)),
                  pl.BlockSpec((bt_tile, h), lambda i, j: (j, 0))],
        out_specs=pl.BlockSpec((bt, h), lambda i, j: (i, 0)),
        out_shape=jax.ShapeDtypeStruct((t, h), jnp.float32),
        scratch_shapes=[pltpu.VMEM((bt, h), jnp.float32)],
        interpret=interpret,
    )(rank, eidx, wts, meta, ys)

    out = pl.pallas_call(
        functools.partial(_shared_body, nfs=nfs),
        grid=(nt, nfs),
        in_specs=[pl.BlockSpec((bt, h), lambda i, j: (i, 0)),
                  pl.BlockSpec((bfs, h), lambda i, j: (j, 0)),
                  pl.BlockSpec((bfs, h), lambda i, j: (j, 0)),
                  pl.BlockSpec((h, bfs), lambda i, j: (0, j)),
                  pl.BlockSpec((bt, h), lambda i, j: (i, 0))],
        out_specs=pl.BlockSpec((bt, h), lambda i, j: (i, 0)),
        out_shape=jax.ShapeDtypeStruct((t, h), jnp.float32),
        scratch_shapes=[pltpu.VMEM((bt, h), jnp.float32)],
        interpret=interpret,
    )(x, Wg_s, Wu_s, Wd_s, comb)

    return out.reshape(b, s, h)


def kernel(hidden_states, gate_w, Wg, Wu, Wd, Wg_s, Wu_s, Wd_s):
    return _impl(hidden_states, gate_w, Wg, Wu, Wd, Wg_s, Wu_s, Wd_s)


# speedup vs baseline: 2.7442x; 2.7442x over previous
import functools

import jax
import jax.numpy as jnp
from jax.experimental import pallas as pl
from jax.experimental.pallas import tpu as pltpu

_SCALE = 1.0


def _fiota(shape, dim):
    return jax.lax.broadcasted_iota(jnp.int32, shape, dim).astype(jnp.float32)


def _route_body(x_ref, gw_ref, w_ref, e_ref):
    x = x_ref[...]
    logits = jax.lax.dot_general(x, gw_ref[...], (((1,), (1,)), ((), ())),
                                 preferred_element_type=jnp.float32)
    m = jnp.max(logits, axis=1, keepdims=True)
    p = jnp.exp(logits - m)
    sc = p / jnp.sum(p, axis=1, keepdims=True)
    bt, ne = sc.shape
    ie = _fiota((bt, ne), 1)
    m1 = jnp.max(sc, axis=1, keepdims=True)
    i1 = jnp.min(jnp.where(sc == m1, ie, float(ne)), axis=1, keepdims=True)
    sc2 = jnp.where(ie == i1, -1.0, sc)
    m2 = jnp.max(sc2, axis=1, keepdims=True)
    i2 = jnp.min(jnp.where(sc2 == m2, ie, float(ne)), axis=1, keepdims=True)
    w_ref[...] = jnp.where(ie == 0.0, m1 * _SCALE,
                           jnp.where(ie == 1.0, m2 * _SCALE, 0.0))
    e_ref[...] = jnp.where(ie == 0.0, i1, jnp.where(ie == 1.0, i2, 0.0))


def _count_body(e_ref, rank_ref, meta_ref, c_ref, *, nt, bt_tile, ntiles):
    b = pl.program_id(0)
    bt, ne = e_ref.shape

    @pl.when(b == 0)
    def _():
        c_ref[...] = jnp.zeros_like(c_ref)

    ie = _fiota((bt, ne), 1)
    o1 = (ie == e_ref[:, 0:1]).astype(jnp.float32)
    o2 = (ie == e_ref[:, 1:2]).astype(jnp.float32)
    s = o1 + o2
    r = _fiota((bt, bt), 0)
    cc = _fiota((bt, bt), 1)
    tri = (r > cc).astype(jnp.float32)
    cum = jnp.dot(tri, s, preferred_element_type=jnp.float32)
    crow = c_ref[0:1, :ne]
    rank1 = jnp.sum(o1 * (cum + crow), axis=1, keepdims=True)
    rank2 = jnp.sum(o2 * (cum + o1 + crow), axis=1, keepdims=True)
    rank_ref[...] = jnp.where(ie == 0.0, rank1,
                              jnp.where(ie == 1.0, rank2, 0.0))
    cnew = crow + jnp.sum(s, axis=0, keepdims=True)
    c_ref[0:1, :ne] = cnew

    @pl.when(b == nt - 1)
    def _():
        pc = jnp.ceil(cnew / bt_tile) * bt_tile
        iu = _fiota((ne, ne), 0)
        ju = _fiota((ne, ne), 1)
        ut = (iu < ju).astype(jnp.float32)
        offs = jnp.dot(pc, ut, preferred_element_type=jnp.float32)
        ends = offs + pc
        rows, cols = meta_ref.shape
        icol = _fiota((1, cols), 1)
        te = jnp.zeros((1, cols), jnp.float32)
        for eidx in range(ne):
            te = te + (icol * bt_tile >= ends[0, eidx]).astype(jnp.float32)
        te = jnp.minimum(te, float(ne - 1))
        irow = _fiota((rows, cols), 0)
        meta_ref[...] = jnp.where(
            irow == 0.0, jnp.pad(cnew, ((0, 0), (0, cols - ne))),
            jnp.where(irow == 1.0, jnp.pad(offs, ((0, 0), (0, cols - ne))),
                      jnp.where(irow == 2.0, te, 0.0)))


def _pos_from(rank, eidx, meta, ne):
    offs = meta[1:2, :ne]
    ie = _fiota(eidx.shape, 1)
    o1 = (ie == eidx[:, 0:1]).astype(jnp.float32)
    o2 = (ie == eidx[:, 1:2]).astype(jnp.float32)
    pos1 = rank[:, 0] + jnp.sum(o1 * offs, axis=1)
    pos2 = rank[:, 1] + jnp.sum(o2 * offs, axis=1)
    return pos1, pos2


def _gather_body(rank_ref, e_ref, meta_ref, x_ref, xs_ref, acc_ref,
                 *, bt_tile, ne, nt):
    i = pl.program_id(0)
    j = pl.program_id(1)
    pos1, pos2 = _pos_from(rank_ref[...], e_ref[...], meta_ref[...], ne)
    bt = rank_ref.shape[0]
    pj = (i * bt_tile
          + _fiota((bt, bt_tile), 1))
    mt = ((pos1[:, None] == pj).astype(jnp.float32)
          + (pos2[:, None] == pj).astype(jnp.float32))
    y = jax.lax.dot_general(mt, x_ref[...], (((0,), (0,)), ((), ())),
                            preferred_element_type=jnp.float32)

    @pl.when(j == 0)
    def _():
        acc_ref[...] = jnp.zeros_like(acc_ref)

    acc_ref[...] += y

    @pl.when(j == nt - 1)
    def _():
        xs_ref[...] = acc_ref[...]


def _gmm_body(te_ref, xs_ref, wg_ref, wu_ref, wd_ref, ys_ref, acc_ref, *, nf):
    f = pl.program_id(1)
    xs = xs_ref[...]
    g = jax.lax.dot_general(xs, wg_ref[0], (((1,), (1,)), ((), ())),
                            preferred_element_type=jnp.float32)
    u = jax.lax.dot_general(xs, wu_ref[0], (((1,), (1,)), ((), ())),
                            preferred_element_type=jnp.float32)
    h = (g * jax.lax.logistic(g)) * u
    y = jax.lax.dot_general(h, wd_ref[0], (((1,), (1,)), ((), ())),
                            preferred_element_type=jnp.float32)

    @pl.when(f == 0)
    def _():
        acc_ref[...] = jnp.zeros_like(acc_ref)

    acc_ref[...] += y

    @pl.when(f == nf - 1)
    def _():
        ys_ref[...] = acc_ref[...]


def _comb_body(rank_ref, e_ref, w_ref, meta_ref, ys_ref, out_ref, acc_ref,
               *, np_chunks, bt_tile, ne):
    p = pl.program_id(1)
    pos1, pos2 = _pos_from(rank_ref[...], e_ref[...], meta_ref[...], ne)
    w = w_ref[...]
    bt = rank_ref.shape[0]
    pj = (p * bt_tile
          + _fiota((bt, bt_tile), 1))
    c = (w[:, 0:1] * (pos1[:, None] == pj).astype(jnp.float32)
         + w[:, 1:2] * (pos2[:, None] == pj).astype(jnp.float32))
    y = jnp.dot(c, ys_ref[...], preferred_element_type=jnp.float32)

    @pl.when(p == 0)
    def _():
        acc_ref[...] = jnp.zeros_like(acc_ref)

    acc_ref[...] += y

    @pl.when(p == np_chunks - 1)
    def _():
        out_ref[...] = acc_ref[...]


def _shared_body(x_ref, wg_ref, wu_ref, wd_ref, comb_ref, out_ref, acc_ref,
                 *, nfs):
    f = pl.program_id(1)
    x = x_ref[...]
    g = jax.lax.dot_general(x, wg_ref[...], (((1,), (1,)), ((), ())),
                            preferred_element_type=jnp.float32)
    u = jax.lax.dot_general(x, wu_ref[...], (((1,), (1,)), ((), ())),
                            preferred_element_type=jnp.float32)
    h = (g * jax.lax.logistic(g)) * u
    y = jax.lax.dot_general(h, wd_ref[...], (((1,), (1,)), ((), ())),
                            preferred_element_type=jnp.float32)

    @pl.when(f == 0)
    def _():
        acc_ref[...] = comb_ref[...]

    acc_ref[...] += y

    @pl.when(f == nfs - 1)
    def _():
        out_ref[...] = acc_ref[...]


def _impl(hidden_states, gate_w, Wg, Wu, Wd, Wg_s, Wu_s, Wd_s,
          bt=256, bt_tile=256, bf=128, bfs=256, interpret=False):
    b, s, h = hidden_states.shape
    t = b * s
    ne, f, _ = Wg.shape
    fs = Wg_s.shape[0]
    x = hidden_states.reshape(t, h).astype(jnp.float32)
    nt = t // bt
    ntiles = (2 * t) // bt_tile + ne - 1
    p_tot = ntiles * bt_tile
    nf = f // bf
    nfs = fs // bfs

    wts, eidx = pl.pallas_call(
        _route_body,
        grid=(nt,),
        in_specs=[pl.BlockSpec((bt, h), lambda i: (i, 0)),
                  pl.BlockSpec((ne, h), lambda i: (0, 0))],
        out_specs=[pl.BlockSpec((bt, ne), lambda i: (i, 0)),
                   pl.BlockSpec((bt, ne), lambda i: (i, 0))],
        out_shape=[jax.ShapeDtypeStruct((t, ne), jnp.float32),
                   jax.ShapeDtypeStruct((t, ne), jnp.float32)],
        interpret=interpret,
    )(x, gate_w)

    rank, meta = pl.pallas_call(
        functools.partial(_count_body, nt=nt, bt_tile=bt_tile, ntiles=ntiles),
        grid=(nt,),
        in_specs=[pl.BlockSpec((bt, ne), lambda i: (i, 0))],
        out_specs=[pl.BlockSpec((bt, ne), lambda i: (i, 0)),
                   pl.BlockSpec((8, 128), lambda i: (0, 0))],
        out_shape=[jax.ShapeDtypeStruct((t, ne), jnp.float32),
                   jax.ShapeDtypeStruct((8, 128), jnp.float32)],
        scratch_shapes=[pltpu.VMEM((8, 128), jnp.float32)],
        interpret=interpret,
    )(eidx)

    tile_e = jnp.round(meta[2, :ntiles]).astype(jnp.int32)

    xs = pl.pallas_call(
        functools.partial(_gather_body, bt_tile=bt_tile, ne=ne, nt=nt),
        grid=(ntiles, nt),
        in_specs=[pl.BlockSpec((bt, ne), lambda i, j: (j, 0)),
                  pl.BlockSpec((bt, ne), lambda i, j: (j, 0)),
                  pl.BlockSpec((8, 128), lambda i, j: (0, 0)),
                  pl.BlockSpec((bt, h), lambda i, j: (j, 0))],
        out_specs=pl.BlockSpec((bt_tile, h), lambda i, j: (i, 0)),
        out_shape=jax.ShapeDtypeStruct((p_tot, h), jnp.float32),
        scratch_shapes=[pltpu.VMEM((bt_tile, h), jnp.float32)],
        interpret=interpret,
    )(rank, eidx, meta, x)

    ys = pl.pallas_call(
        functools.partial(_gmm_body, nf=nf),
        grid_spec=pltpu.PrefetchScalarGridSpec(
            num_scalar_prefetch=1,
            grid=(ntiles, nf),
            in_specs=[
                pl.BlockSpec((bt_tile, h), lambda i, j, te: (i, 0)),
                pl.BlockSpec((1, bf, h), lambda i, j, te: (te[i], j, 0)),
                pl.BlockSpec((1, bf, h), lambda i, j, te: (te[i], j, 0)),
                pl.BlockSpec((1, h, bf), lambda i, j, te: (te[i], 0, j)),
            ],
            out_specs=pl.BlockSpec((bt_tile, h), lambda i, j, te: (i, 0)),
            scratch_shapes=[pltpu.VMEM((bt_tile, h), jnp.float32)],
        ),
        out_shape=jax.ShapeDtypeStruct((p_tot, h), jnp.float32),
        interpret=interpret,
    )(tile_e, xs, Wg, Wu, Wd)

    comb = pl.pallas_call(
        functools.partial(_comb_body, np_chunks=ntiles, bt_tile=bt_tile,
                          ne=ne),
        grid=(nt, ntiles),
        in_specs=[pl.BlockSpec((bt, ne), lambda i, j: (i, 0)),
                  pl.BlockSpec((bt, ne), lambda i, j: (i, 0)),
                  pl.BlockSpec((bt, ne), lambda i, j: (i, 0)),
                  pl.BlockSpec((8, 128), lambda i, j: (0, 0)),
                  pl.BlockSpec((bt_tile, h), lambda i, j: (j, 0))],
        out_specs=pl.BlockSpec((bt, h), lambda i, j: (i, 0)),
        out_shape=jax.ShapeDtypeStruct((t, h), jnp.float32),
        scratch_shapes=[pltpu.VMEM((bt, h), jnp.float32)],
        interpret=interpret,
    )(rank, eidx, wts, meta, ys)

    out = pl.pallas_call(
        functools.partial(_shared_body, nfs=nfs),
        grid=(nt, nfs),
        in_specs=[pl.BlockSpec((bt, h), lambda i, j: (i, 0)),
                  pl.BlockSpec((bfs, h), lambda i, j: (j, 0)),
                  pl.BlockSpec((bfs, h), lambda i, j: (j, 0)),
                  pl.BlockSpec((h, bfs), lambda i, j: (0, j)),
                  pl.BlockSpec((bt, h), lambda i, j: (i, 0))],
        out_specs=pl.BlockSpec((bt, h), lambda i, j: (i, 0)),
        out_shape=jax.ShapeDtypeStruct((t, h), jnp.float32),
        scratch_shapes=[pltpu.VMEM((bt, h), jnp.float32)],
        interpret=interpret,
    )(x, Wg_s, Wu_s, Wd_s, comb)

    return out.reshape(b, s, h)


def kernel(hidden_states, gate_w, Wg, Wu, Wd, Wg_s, Wu_s, Wd_s):
    return _impl(hidden_states, gate_w, Wg, Wu, Wd, Wg_s, Wu_s, Wd_s)


# speedup vs baseline: 5.7072x; 2.0797x over previous
import functools

import jax
import jax.numpy as jnp
from jax import lax
from jax.experimental import pallas as pl
from jax.experimental.pallas import tpu as pltpu
from jax.experimental.pallas import tpu_sc as plsc

_SCALE = 1.0


def _fiota(shape, dim):
    return jax.lax.broadcasted_iota(jnp.int32, shape, dim).astype(jnp.float32)


def _route_body(x_ref, gw_ref, w_ref, e_ref):
    x = x_ref[...]
    logits = jax.lax.dot_general(x, gw_ref[...], (((1,), (1,)), ((), ())),
                                 preferred_element_type=jnp.float32)
    m = jnp.max(logits, axis=1, keepdims=True)
    p = jnp.exp(logits - m)
    sc = p / jnp.sum(p, axis=1, keepdims=True)
    bt, ne = sc.shape
    ie = _fiota((bt, ne), 1)
    m1 = jnp.max(sc, axis=1, keepdims=True)
    i1 = jnp.min(jnp.where(sc == m1, ie, float(ne)), axis=1, keepdims=True)
    sc2 = jnp.where(ie == i1, -1.0, sc)
    m2 = jnp.max(sc2, axis=1, keepdims=True)
    i2 = jnp.min(jnp.where(sc2 == m2, ie, float(ne)), axis=1, keepdims=True)
    w_ref[...] = jnp.where(ie == 0.0, m1 * _SCALE,
                           jnp.where(ie == 1.0, m2 * _SCALE, 0.0))
    e_ref[...] = jnp.where(ie == 0.0, i1, jnp.where(ie == 1.0, i2, 0.0))


def _count_body(e_ref, rank_ref, meta_ref, c_ref, *, nt, bt_tile, ntiles):
    b = pl.program_id(0)
    bt, ne = e_ref.shape

    @pl.when(b == 0)
    def _():
        c_ref[...] = jnp.zeros_like(c_ref)

    ie = _fiota((bt, ne), 1)
    o1 = (ie == e_ref[:, 0:1]).astype(jnp.float32)
    o2 = (ie == e_ref[:, 1:2]).astype(jnp.float32)
    s = o1 + o2
    r = _fiota((bt, bt), 0)
    cc = _fiota((bt, bt), 1)
    tri = (r > cc).astype(jnp.float32)
    cum = jnp.dot(tri, s, preferred_element_type=jnp.float32)
    crow = c_ref[0:1, :ne]
    rank1 = jnp.sum(o1 * (cum + crow), axis=1, keepdims=True)
    rank2 = jnp.sum(o2 * (cum + o1 + crow), axis=1, keepdims=True)
    rank_ref[...] = jnp.where(ie == 0.0, rank1,
                              jnp.where(ie == 1.0, rank2, 0.0))
    cnew = crow + jnp.sum(s, axis=0, keepdims=True)
    c_ref[0:1, :ne] = cnew

    @pl.when(b == nt - 1)
    def _():
        pc = jnp.ceil(cnew / bt_tile) * bt_tile
        iu = _fiota((ne, ne), 0)
        ju = _fiota((ne, ne), 1)
        ut = (iu < ju).astype(jnp.float32)
        offs = jnp.dot(pc, ut, preferred_element_type=jnp.float32)
        ends = offs + pc
        rows, cols = meta_ref.shape
        icol = _fiota((1, cols), 1)
        te = jnp.zeros((1, cols), jnp.float32)
        for eidx in range(ne):
            te = te + (icol * bt_tile >= ends[0, eidx]).astype(jnp.float32)
        te = jnp.minimum(te, float(ne - 1))
        irow = _fiota((rows, cols), 0)
        na = ends[0, ne - 1] / bt_tile
        meta_ref[...] = jnp.where(
            irow == 0.0, jnp.pad(cnew, ((0, 0), (0, cols - ne))),
            jnp.where(irow == 1.0, jnp.pad(offs, ((0, 0), (0, cols - ne))),
                      jnp.where(irow == 2.0, te,
                                jnp.where(irow == 3.0, na, 0.0))))


def _pos_from(rank, eidx, meta, ne):
    offs = meta[1:2, :ne]
    ie = _fiota(eidx.shape, 1)
    o1 = (ie == eidx[:, 0:1]).astype(jnp.float32)
    o2 = (ie == eidx[:, 1:2]).astype(jnp.float32)
    pos1 = rank[:, 0] + jnp.sum(o1 * offs, axis=1)
    pos2 = rank[:, 1] + jnp.sum(o2 * offs, axis=1)
    return pos1, pos2


def _posw_body(rank_ref, e_ref, w_ref, meta_ref, posw_ref, *, ne):
    pos1, pos2 = _pos_from(rank_ref[...], e_ref[...], meta_ref[...], ne)
    w = w_ref[...]
    ie = _fiota(w.shape, 1)
    posw_ref[...] = jnp.where(
        ie == 0.0, pos1[:, None],
        jnp.where(ie == 1.0, pos2[:, None],
                  jnp.where(ie == 2.0, w[:, 0:1],
                            jnp.where(ie == 3.0, w[:, 1:2], 0.0))))


def _sc_build_xs(x, pos1, pos2, p_tot):
    t, h = x.shape
    info = plsc.get_sparse_core_info()
    nw = info.num_cores * info.num_subcores
    chunk = 16
    nchunks = t // (nw * chunk)
    mesh = plsc.VectorSubcoreMesh(core_axis_name="c", subcore_axis_name="s")

    @functools.partial(
        pl.kernel, mesh=mesh,
        out_type=jax.ShapeDtypeStruct((p_tot, h), jnp.float32),
        scratch_types=[pltpu.VMEM((chunk,), jnp.int32),
                       pltpu.VMEM((chunk,), jnp.int32),
                       pltpu.VMEM((chunk, h), jnp.float32),
                       pltpu.SemaphoreType.DMA],
    )
    def k(x_hbm, p1_hbm, p2_hbm, out_hbm, i1_v, i2_v, rows_v, sem):
        wid = lax.axis_index("s") * info.num_cores + lax.axis_index("c")
        for c in range(nchunks):
            base = wid * (nchunks * chunk) + c * chunk
            pltpu.sync_copy(p1_hbm.at[pl.ds(base, chunk)], i1_v)
            pltpu.sync_copy(p2_hbm.at[pl.ds(base, chunk)], i2_v)
            pltpu.sync_copy(x_hbm.at[pl.ds(base, chunk)], rows_v)
            pltpu.async_copy(rows_v, out_hbm.at[i1_v], sem).wait()
            pltpu.async_copy(rows_v, out_hbm.at[i2_v], sem).wait()

    return k(x, pos1, pos2)


def _sc_gather_ys(ys, pos1, pos2, t):
    h = ys.shape[1]
    info = plsc.get_sparse_core_info()
    nw = info.num_cores * info.num_subcores
    chunk = 16
    nchunks = t // (nw * chunk)
    mesh = plsc.VectorSubcoreMesh(core_axis_name="c", subcore_axis_name="s")

    @functools.partial(
        pl.kernel, mesh=mesh,
        out_type=[jax.ShapeDtypeStruct((t, h), jnp.float32),
                  jax.ShapeDtypeStruct((t, h), jnp.float32)],
        scratch_types=[pltpu.VMEM((chunk,), jnp.int32),
                       pltpu.VMEM((chunk,), jnp.int32),
                       pltpu.VMEM((chunk, h), jnp.float32),
                       pltpu.VMEM((chunk, h), jnp.float32),
                       pltpu.SemaphoreType.DMA],
    )
    def k(ys_hbm, p1_hbm, p2_hbm, y1_hbm, y2_hbm, i1_v, i2_v, r1_v, r2_v,
          sem):
        wid = lax.axis_index("s") * info.num_cores + lax.axis_index("c")
        for c in range(nchunks):
            base = wid * (nchunks * chunk) + c * chunk
            pltpu.sync_copy(p1_hbm.at[pl.ds(base, chunk)], i1_v)
            pltpu.sync_copy(p2_hbm.at[pl.ds(base, chunk)], i2_v)
            pltpu.async_copy(ys_hbm.at[i1_v], r1_v, sem).wait()
            pltpu.async_copy(ys_hbm.at[i2_v], r2_v, sem).wait()
            pltpu.sync_copy(r1_v, y1_hbm.at[pl.ds(base, chunk)])
            pltpu.sync_copy(r2_v, y2_hbm.at[pl.ds(base, chunk)])

    return k(ys, pos1, pos2)


def _gmm_body(te_ref, na_ref, xs_ref, wg_ref, wu_ref, wd_ref, ys_ref,
              acc_ref, *, nf):
    i = pl.program_id(0)
    f = pl.program_id(1)

    @pl.when(i < na_ref[0])
    def _():
        _gmm_active(xs_ref, wg_ref, wu_ref, wd_ref, ys_ref, acc_ref, f, nf)


def _gmm_active(xs_ref, wg_ref, wu_ref, wd_ref, ys_ref, acc_ref, f, nf):
    xs = xs_ref[...]
    g = jax.lax.dot_general(xs, wg_ref[0], (((1,), (1,)), ((), ())),
                            preferred_element_type=jnp.float32)
    u = jax.lax.dot_general(xs, wu_ref[0], (((1,), (1,)), ((), ())),
                            preferred_element_type=jnp.float32)
    h = (g * jax.lax.logistic(g)) * u
    y = jax.lax.dot_general(h, wd_ref[0], (((1,), (1,)), ((), ())),
                            preferred_element_type=jnp.float32)

    @pl.when(f == 0)
    def _():
        acc_ref[...] = jnp.zeros_like(acc_ref)

    acc_ref[...] += y

    @pl.when(f == nf - 1)
    def _():
        ys_ref[...] = acc_ref[...]


def _shared_body(x_ref, wg_ref, wu_ref, wd_ref, w_ref, y1_ref, y2_ref,
                 out_ref, acc_ref, *, nfs):
    f = pl.program_id(1)
    x = x_ref[...]
    g = jax.lax.dot_general(x, wg_ref[...], (((1,), (1,)), ((), ())),
                            preferred_element_type=jnp.float32)
    u = jax.lax.dot_general(x, wu_ref[...], (((1,), (1,)), ((), ())),
                            preferred_element_type=jnp.float32)
    h = (g * jax.lax.logistic(g)) * u
    y = jax.lax.dot_general(h, wd_ref[...], (((1,), (1,)), ((), ())),
                            preferred_element_type=jnp.float32)

    @pl.when(f == 0)
    def _():
        w = w_ref[...]
        acc_ref[...] = w[:, 0:1] * y1_ref[...] + w[:, 1:2] * y2_ref[...]

    acc_ref[...] += y

    @pl.when(f == nfs - 1)
    def _():
        out_ref[...] = acc_ref[...]


def _impl(hidden_states, gate_w, Wg, Wu, Wd, Wg_s, Wu_s, Wd_s,
          bt=256, bt_tile=256, bf=128, bfs=256, interpret=False):
    b, s, h = hidden_states.shape
    t = b * s
    ne, f, _ = Wg.shape
    fs = Wg_s.shape[0]
    x = hidden_states.reshape(t, h).astype(jnp.float32)
    nt = t // bt
    ntiles = (2 * t) // bt_tile + ne - 1
    p_tot = ntiles * bt_tile
    nf = f // bf
    nfs = fs // bfs

    wts, eidx = pl.pallas_call(
        _route_body,
        grid=(nt,),
        in_specs=[pl.BlockSpec((bt, h), lambda i: (i, 0)),
                  pl.BlockSpec((ne, h), lambda i: (0, 0))],
        out_specs=[pl.BlockSpec((bt, ne), lambda i: (i, 0)),
                   pl.BlockSpec((bt, ne), lambda i: (i, 0))],
        out_shape=[jax.ShapeDtypeStruct((t, ne), jnp.float32),
                   jax.ShapeDtypeStruct((t, ne), jnp.float32)],
        interpret=interpret,
    )(x, gate_w)

    rank, meta = pl.pallas_call(
        functools.partial(_count_body, nt=nt, bt_tile=bt_tile, ntiles=ntiles),
        grid=(nt,),
        in_specs=[pl.BlockSpec((bt, ne), lambda i: (i, 0))],
        out_specs=[pl.BlockSpec((bt, ne), lambda i: (i, 0)),
                   pl.BlockSpec((8, 128), lambda i: (0, 0))],
        out_shape=[jax.ShapeDtypeStruct((t, ne), jnp.float32),
                   jax.ShapeDtypeStruct((8, 128), jnp.float32)],
        scratch_shapes=[pltpu.VMEM((8, 128), jnp.float32)],
        interpret=interpret,
    )(eidx)

    tile_e = jnp.round(meta[2, :ntiles]).astype(jnp.int32)
    na_arr = jnp.round(meta[3:4, 0]).astype(jnp.int32)

    posw = pl.pallas_call(
        functools.partial(_posw_body, ne=ne),
        grid=(nt,),
        in_specs=[pl.BlockSpec((bt, ne), lambda i: (i, 0)),
                  pl.BlockSpec((bt, ne), lambda i: (i, 0)),
                  pl.BlockSpec((bt, ne), lambda i: (i, 0)),
                  pl.BlockSpec((8, 128), lambda i: (0, 0))],
        out_specs=pl.BlockSpec((bt, ne), lambda i: (i, 0)),
        out_shape=jax.ShapeDtypeStruct((t, ne), jnp.float32),
        interpret=interpret,
    )(rank, eidx, wts, meta)

    pos1 = posw[:, 0].astype(jnp.int32)
    pos2 = posw[:, 1].astype(jnp.int32)

    xs = _sc_build_xs(x, pos1, pos2, p_tot)

    ys = pl.pallas_call(
        functools.partial(_gmm_body, nf=nf),
        grid_spec=pltpu.PrefetchScalarGridSpec(
            num_scalar_prefetch=2,
            grid=(ntiles, nf),
            in_specs=[
                pl.BlockSpec(
                    (bt_tile, h),
                    lambda i, j, te, na: (jnp.minimum(i, na[0] - 1), 0)),
                pl.BlockSpec(
                    (1, bf, h),
                    lambda i, j, te, na: (te[i],
                                          jnp.where(i < na[0], j, 0), 0)),
                pl.BlockSpec(
                    (1, bf, h),
                    lambda i, j, te, na: (te[i],
                                          jnp.where(i < na[0], j, 0), 0)),
                pl.BlockSpec(
                    (1, h, bf),
                    lambda i, j, te, na: (te[i], 0,
                                          jnp.where(i < na[0], j, 0))),
            ],
            out_specs=pl.BlockSpec((bt_tile, h), lambda i, j, te, na: (i, 0)),
            scratch_shapes=[pltpu.VMEM((bt_tile, h), jnp.float32)],
        ),
        out_shape=jax.ShapeDtypeStruct((p_tot, h), jnp.float32),
        interpret=interpret,
    )(tile_e, na_arr, xs, Wg, Wu, Wd)

    y1, y2 = _sc_gather_ys(ys, pos1, pos2, t)

    out = pl.pallas_call(
        functools.partial(_shared_body, nfs=nfs),
        grid=(nt, nfs),
        in_specs=[pl.BlockSpec((bt, h), lambda i, j: (i, 0)),
                  pl.BlockSpec((bfs, h), lambda i, j: (j, 0)),
                  pl.BlockSpec((bfs, h), lambda i, j: (j, 0)),
                  pl.BlockSpec((h, bfs), lambda i, j: (0, j)),
                  pl.BlockSpec((bt, ne), lambda i, j: (i, 0)),
                  pl.BlockSpec((bt, h), lambda i, j: (i, 0)),
                  pl.BlockSpec((bt, h), lambda i, j: (i, 0))],
        out_specs=pl.BlockSpec((bt, h), lambda i, j: (i, 0)),
        out_shape=jax.ShapeDtypeStruct((t, h), jnp.float32),
        scratch_shapes=[pltpu.VMEM((bt, h), jnp.float32)],
        interpret=interpret,
    )(x, Wg_s, Wu_s, Wd_s, wts, y1, y2)

    return out.reshape(b, s, h)


def kernel(hidden_states, gate_w, Wg, Wu, Wd, Wg_s, Wu_s, Wd_s):
    return _impl(hidden_states, gate_w, Wg, Wu, Wd, Wg_s, Wu_s, Wd_s)
